# baseline (speedup 1.0000x reference)
import functools
import math

import jax
import jax.numpy as jnp
from jax import lax
from jax.experimental import pallas as pl
from jax.experimental.pallas import tpu as pltpu
from jax.experimental.pallas import tpu_sc as plsc

EMBED_DIM = 64
PADDED_DIM = 128
NUM_CORES = 2
NUM_SUBCORES = 16
NUM_WORKERS = NUM_CORES * NUM_SUBCORES
CHUNK = 128
TBLK = 2048

_mesh = plsc.VectorSubcoreMesh(
    core_axis_name="c",
    subcore_axis_name="s",
    num_cores=NUM_CORES,
    num_subcores=NUM_SUBCORES,
)


def _tc_stage(tab_t):
    d, v = tab_t.shape

    def body(in_ref, out_ref):
        out_ref[:, 0:EMBED_DIM] = in_ref[...].T

    return pl.pallas_call(
        body,
        grid=(math.ceil(v / TBLK),),
        in_specs=[pl.BlockSpec((d, TBLK), lambda g: (0, g))],
        out_specs=pl.BlockSpec((TBLK, PADDED_DIM), lambda g: (g, 0)),
        out_shape=jax.ShapeDtypeStruct((v, PADDED_DIM), jnp.float32),
    )(tab_t)


@functools.partial(jax.jit, static_argnames=("seq",))
def _lookup(ids_t, table_p, seq):
    @functools.partial(
        pl.kernel,
        out_type=jax.ShapeDtypeStruct(
            (seq, EMBED_DIM // 8, NUM_WORKERS, 8, CHUNK), jnp.float32
        ),
        mesh=_mesh,
        compiler_params=pltpu.CompilerParams(
            use_tc_tiling_on_sc=False, needs_layout_passes=False
        ),
        scratch_types=[
            pltpu.VMEM((seq, CHUNK), jnp.int32),
            pltpu.VMEM((CHUNK, PADDED_DIM), jnp.float32),
            pltpu.VMEM((CHUNK, PADDED_DIM), jnp.float32),
            pltpu.VMEM((EMBED_DIM // 8, 8, CHUNK), jnp.float32),
            pltpu.SemaphoreType.DMA,
            pltpu.SemaphoreType.DMA,
        ],
    )
    def body(ids_hbm, table_hbm, out_hbm, idx_v, buf0, buf1, buf_t, sem0, sem1):
        wid = lax.axis_index("s") * NUM_CORES + lax.axis_index("c")
        pltpu.sync_copy(ids_hbm.at[:, pl.ds(wid * CHUNK, CHUNK)], idx_v)

        bufs = (buf0, buf1)
        sems = (sem0, sem1)

        def transpose_and_store(b, s):
            iota16 = lax.iota(jnp.int32, 16)
            for r0 in range(0, CHUNK, 16):
                rows = r0 + iota16
                for d in range(EMBED_DIM):
                    lanes = jnp.full((16,), d, jnp.int32)
                    vals = plsc.load_gather(bufs[b], [rows, lanes])
                    buf_t[d // 8, d % 8, pl.ds(r0, 16)] = vals

            pltpu.sync_copy(buf_t, out_hbm.at[s, :, wid])

        pltpu.async_copy(table_hbm.at[idx_v.at[0]], buf0, sem0)
        pltpu.async_copy(table_hbm.at[idx_v.at[1]], buf1, sem1)

        @pl.loop(0, seq - 2, step=2)
        def _(g):
            for b in range(2):
                s = g + b
                pltpu.make_async_copy(
                    table_hbm.at[idx_v.at[s]], bufs[b], sems[b]
                ).wait()
                transpose_and_store(b, s)
                pltpu.async_copy(table_hbm.at[idx_v.at[s + 2]], bufs[b], sems[b])

        for b in range(2):
            s = seq - 2 + b
            pltpu.make_async_copy(
                table_hbm.at[idx_v.at[s]], bufs[b], sems[b]
            ).wait()
            transpose_and_store(b, s)

    return body(ids_t, table_p)


def kernel(token_ids, table):
    batch, seq = token_ids.shape
    assert batch == NUM_WORKERS * CHUNK and seq % 2 == 0
    ids_t = token_ids.T.astype(jnp.int32)
    table_p = _tc_stage(table.T)
    out5 = _lookup(ids_t, table_p, seq)
    return out5.transpose(2, 4, 0, 1, 3).reshape(batch, seq, EMBED_DIM)


# speedup vs baseline: 1.0008x; 1.0008x over previous
import functools
import math

import jax
import jax.numpy as jnp
from jax import lax
from jax.experimental import pallas as pl
from jax.experimental.pallas import tpu as pltpu
from jax.experimental.pallas import tpu_sc as plsc

EMBED_DIM = 64
PADDED_DIM = 128
NUM_CORES = 2
NUM_SUBCORES = 16
NUM_WORKERS = NUM_CORES * NUM_SUBCORES
CHUNK = 128
TBLK = 2048

_mesh = plsc.VectorSubcoreMesh(
    core_axis_name="c",
    subcore_axis_name="s",
    num_cores=NUM_CORES,
    num_subcores=NUM_SUBCORES,
)


def _tc_stage(tab_t):
    d, v = tab_t.shape

    def body(in_ref, out_ref):
        out_ref[:, 0:EMBED_DIM] = in_ref[...].T

    return pl.pallas_call(
        body,
        grid=(math.ceil(v / TBLK),),
        in_specs=[pl.BlockSpec((d, TBLK), lambda g: (0, g))],
        out_specs=pl.BlockSpec((TBLK, PADDED_DIM), lambda g: (g, 0)),
        out_shape=jax.ShapeDtypeStruct((v, PADDED_DIM), jnp.float32),
    )(tab_t)


@functools.partial(jax.jit, static_argnames=("seq",))
def _lookup(ids_t, table_p, seq):
    @functools.partial(
        pl.kernel,
        out_type=jax.ShapeDtypeStruct(
            (seq, EMBED_DIM // 8, NUM_WORKERS, 8, CHUNK), jnp.float32
        ),
        mesh=_mesh,
        compiler_params=pltpu.CompilerParams(
            use_tc_tiling_on_sc=False,
            needs_layout_passes=False,
            disable_bounds_checks=True,
        ),
        scratch_types=[
            pltpu.VMEM((seq, CHUNK), jnp.int32),
            pltpu.VMEM((CHUNK, PADDED_DIM), jnp.float32),
            pltpu.VMEM((CHUNK, PADDED_DIM), jnp.float32),
            pltpu.VMEM((EMBED_DIM // 8, 8, CHUNK), jnp.float32),
            pltpu.SemaphoreType.DMA,
            pltpu.SemaphoreType.DMA,
        ],
    )
    def body(ids_hbm, table_hbm, out_hbm, idx_v, buf0, buf1, buf_t, sem0, sem1):
        wid = lax.axis_index("s") * NUM_CORES + lax.axis_index("c")
        pltpu.sync_copy(ids_hbm.at[:, pl.ds(wid * CHUNK, CHUNK)], idx_v)

        bufs = (buf0, buf1)
        sems = (sem0, sem1)

        def transpose_and_store(b, s):
            iota16 = lax.iota(jnp.int32, 16)
            for r0 in range(0, CHUNK, 16):
                rows = r0 + iota16
                for d in range(EMBED_DIM):
                    lanes = jnp.full((16,), d, jnp.int32)
                    vals = plsc.load_gather(bufs[b], [rows, lanes])
                    buf_t[d // 8, d % 8, pl.ds(r0, 16)] = vals

            pltpu.sync_copy(buf_t, out_hbm.at[s, :, wid])

        pltpu.async_copy(table_hbm.at[idx_v.at[0]], buf0, sem0)
        pltpu.async_copy(table_hbm.at[idx_v.at[1]], buf1, sem1)

        @pl.loop(0, seq - 2, step=2)
        def _(g):
            for b in range(2):
                s = g + b
                pltpu.make_async_copy(
                    table_hbm.at[idx_v.at[s]], bufs[b], sems[b]
                ).wait()
                transpose_and_store(b, s)
                pltpu.async_copy(table_hbm.at[idx_v.at[s + 2]], bufs[b], sems[b])

        for b in range(2):
            s = seq - 2 + b
            pltpu.make_async_copy(
                table_hbm.at[idx_v.at[s]], bufs[b], sems[b]
            ).wait()
            transpose_and_store(b, s)

    return body(ids_t, table_p)


def kernel(token_ids, table):
    batch, seq = token_ids.shape
    assert batch == NUM_WORKERS * CHUNK and seq % 2 == 0
    ids_t = token_ids.T.astype(jnp.int32)
    table_p = _tc_stage(table.T)
    out5 = _lookup(ids_t, table_p, seq)
    return out5.transpose(2, 4, 0, 1, 3).reshape(batch, seq, EMBED_DIM)


# speedup vs baseline: 1.7848x; 1.7834x over previous
import functools
import math

import jax
import jax.numpy as jnp
from jax import lax
from jax.experimental import pallas as pl
from jax.experimental.pallas import tpu as pltpu
from jax.experimental.pallas import tpu_sc as plsc

EMBED_DIM = 64
PADDED_DIM = 128
NUM_CORES = 2
NUM_SUBCORES = 16
NUM_WORKERS = NUM_CORES * NUM_SUBCORES
CHUNK = 128
TBLK = 8192

_mesh = plsc.VectorSubcoreMesh(
    core_axis_name="c",
    subcore_axis_name="s",
    num_cores=NUM_CORES,
    num_subcores=NUM_SUBCORES,
)


def _tc_stage(tab_t):
    d, v = tab_t.shape

    def body(in_ref, out_ref):
        out_ref[:, 0:EMBED_DIM] = in_ref[...].T

    return pl.pallas_call(
        body,
        grid=(math.ceil(v / TBLK),),
        in_specs=[pl.BlockSpec((d, TBLK), lambda g: (0, g))],
        out_specs=pl.BlockSpec((TBLK, PADDED_DIM), lambda g: (g, 0)),
        out_shape=jax.ShapeDtypeStruct((v, PADDED_DIM), jnp.float32),
    )(tab_t)


@functools.partial(jax.jit, static_argnames=("n_chunks",))
def _lookup(ids, table_p, n_chunks):
    @functools.partial(
        pl.kernel,
        out_type=jax.ShapeDtypeStruct(
            (NUM_WORKERS, n_chunks, CHUNK, EMBED_DIM), jnp.float32
        ),
        mesh=_mesh,
        compiler_params=pltpu.CompilerParams(use_tc_tiling_on_sc=False),
        scratch_types=[
            pltpu.VMEM((n_chunks, CHUNK), jnp.int32),
            pltpu.VMEM((CHUNK, PADDED_DIM), jnp.float32),
            pltpu.VMEM((CHUNK, PADDED_DIM), jnp.float32),
            pltpu.SemaphoreType.DMA,
            pltpu.SemaphoreType.DMA,
        ],
    )
    def body(ids_hbm, table_hbm, out_hbm, idx_v, buf0, buf1, sem0, sem1):
        wid = lax.axis_index("s") * NUM_CORES + lax.axis_index("c")
        pltpu.sync_copy(ids_hbm.at[wid], idx_v)

        bufs = (buf0, buf1)
        sems = (sem0, sem1)

        pltpu.async_copy(table_hbm.at[idx_v.at[0]], buf0, sem0)
        pltpu.async_copy(table_hbm.at[idx_v.at[1]], buf1, sem1)

        @pl.loop(0, n_chunks - 2, step=2)
        def _(g):
            for b in range(2):
                j = g + b
                pltpu.make_async_copy(
                    table_hbm.at[idx_v.at[j]], bufs[b], sems[b]
                ).wait()
                pltpu.sync_copy(
                    bufs[b].at[:, pl.ds(0, EMBED_DIM)], out_hbm.at[wid, j]
                )
                pltpu.async_copy(table_hbm.at[idx_v.at[j + 2]], bufs[b], sems[b])

        for b in range(2):
            j = n_chunks - 2 + b
            pltpu.make_async_copy(
                table_hbm.at[idx_v.at[j]], bufs[b], sems[b]
            ).wait()
            pltpu.sync_copy(
                bufs[b].at[:, pl.ds(0, EMBED_DIM)], out_hbm.at[wid, j]
            )

    return body(ids, table_p)


def kernel(token_ids, table):
    batch, seq = token_ids.shape
    total = batch * seq
    assert total % (NUM_WORKERS * CHUNK) == 0
    n_chunks = total // (NUM_WORKERS * CHUNK)
    ids = token_ids.reshape(NUM_WORKERS, n_chunks, CHUNK).astype(jnp.int32)
    table_p = _tc_stage(table.T)
    out = _lookup(ids, table_p, n_chunks)
    return out.reshape(batch, seq, EMBED_DIM)


# speedup vs baseline: 1.8611x; 1.0427x over previous
import functools
import math

import jax
import jax.numpy as jnp
from jax import lax
from jax.experimental import pallas as pl
from jax.experimental.pallas import tpu as pltpu
from jax.experimental.pallas import tpu_sc as plsc

EMBED_DIM = 64
PADDED_DIM = 128
NUM_CORES = 2
NUM_SUBCORES = 16
NUM_WORKERS = NUM_CORES * NUM_SUBCORES
CHUNK = 128
TBLK = 16384

_mesh = plsc.VectorSubcoreMesh(
    core_axis_name="c",
    subcore_axis_name="s",
    num_cores=NUM_CORES,
    num_subcores=NUM_SUBCORES,
)


def _tc_stage(tab_t):
    d, v = tab_t.shape

    def body(in_ref, out_ref):
        out_ref[:, 0:EMBED_DIM] = in_ref[...].T

    return pl.pallas_call(
        body,
        grid=(math.ceil(v / TBLK),),
        in_specs=[pl.BlockSpec((d, TBLK), lambda g: (0, g))],
        out_specs=pl.BlockSpec((TBLK, PADDED_DIM), lambda g: (g, 0)),
        out_shape=jax.ShapeDtypeStruct((v, PADDED_DIM), jnp.float32),
    )(tab_t)


@functools.partial(jax.jit, static_argnames=("n_chunks",))
def _lookup(ids, table_p, n_chunks):
    @functools.partial(
        pl.kernel,
        out_type=jax.ShapeDtypeStruct(
            (NUM_WORKERS, n_chunks, CHUNK, EMBED_DIM), jnp.float32
        ),
        mesh=_mesh,
        compiler_params=pltpu.CompilerParams(use_tc_tiling_on_sc=False),
        scratch_types=[
            pltpu.VMEM((n_chunks, CHUNK), jnp.int32),
            pltpu.VMEM((CHUNK, PADDED_DIM), jnp.float32),
            pltpu.VMEM((CHUNK, PADDED_DIM), jnp.float32),
            pltpu.SemaphoreType.DMA,
            pltpu.SemaphoreType.DMA,
        ],
    )
    def body(ids_hbm, table_hbm, out_hbm, idx_v, buf0, buf1, sem0, sem1):
        wid = lax.axis_index("s") * NUM_CORES + lax.axis_index("c")
        pltpu.sync_copy(ids_hbm.at[wid], idx_v)

        bufs = (buf0, buf1)
        sems = (sem0, sem1)

        pltpu.async_copy(table_hbm.at[idx_v.at[0]], buf0, sem0)
        pltpu.async_copy(table_hbm.at[idx_v.at[1]], buf1, sem1)

        @pl.loop(0, n_chunks - 2, step=2)
        def _(g):
            for b in range(2):
                j = g + b
                pltpu.make_async_copy(
                    table_hbm.at[idx_v.at[j]], bufs[b], sems[b]
                ).wait()
                pltpu.sync_copy(
                    bufs[b].at[:, pl.ds(0, EMBED_DIM)], out_hbm.at[wid, j]
                )
                pltpu.async_copy(table_hbm.at[idx_v.at[j + 2]], bufs[b], sems[b])

        for b in range(2):
            j = n_chunks - 2 + b
            pltpu.make_async_copy(
                table_hbm.at[idx_v.at[j]], bufs[b], sems[b]
            ).wait()
            pltpu.sync_copy(
                bufs[b].at[:, pl.ds(0, EMBED_DIM)], out_hbm.at[wid, j]
            )

    return body(ids, table_p)


def kernel(token_ids, table):
    batch, seq = token_ids.shape
    total = batch * seq
    assert total % (NUM_WORKERS * CHUNK) == 0
    n_chunks = total // (NUM_WORKERS * CHUNK)
    ids = token_ids.reshape(NUM_WORKERS, n_chunks, CHUNK).astype(jnp.int32)
    table_p = _tc_stage(table.T)
    out = _lookup(ids, table_p, n_chunks)
    return out.reshape(batch, seq, EMBED_DIM)


# speedup vs baseline: 1.8877x; 1.0143x over previous
import functools
import math

import jax
import jax.numpy as jnp
from jax import lax
from jax.experimental import pallas as pl
from jax.experimental.pallas import tpu as pltpu
from jax.experimental.pallas import tpu_sc as plsc

EMBED_DIM = 64
PADDED_DIM = 128
NUM_CORES = 2
NUM_SUBCORES = 16
NUM_WORKERS = NUM_CORES * NUM_SUBCORES
CHUNK = 128
TBLK = 32768

_mesh = plsc.VectorSubcoreMesh(
    core_axis_name="c",
    subcore_axis_name="s",
    num_cores=NUM_CORES,
    num_subcores=NUM_SUBCORES,
)


def _tc_stage(tab_t):
    d, v = tab_t.shape

    def body(in_ref, out_ref):
        out_ref[:, 0:EMBED_DIM] = in_ref[...].T

    return pl.pallas_call(
        body,
        grid=(math.ceil(v / TBLK),),
        in_specs=[pl.BlockSpec((d, TBLK), lambda g: (0, g))],
        out_specs=pl.BlockSpec((TBLK, PADDED_DIM), lambda g: (g, 0)),
        out_shape=jax.ShapeDtypeStruct((v, PADDED_DIM), jnp.float32),
    )(tab_t)


@functools.partial(jax.jit, static_argnames=("n_chunks",))
def _lookup(ids, table_p, n_chunks):
    @functools.partial(
        pl.kernel,
        out_type=jax.ShapeDtypeStruct(
            (NUM_WORKERS, n_chunks, CHUNK, EMBED_DIM), jnp.float32
        ),
        mesh=_mesh,
        compiler_params=pltpu.CompilerParams(use_tc_tiling_on_sc=False),
        scratch_types=[
            pltpu.VMEM((n_chunks, CHUNK), jnp.int32),
            pltpu.VMEM((CHUNK, PADDED_DIM), jnp.float32),
            pltpu.VMEM((CHUNK, PADDED_DIM), jnp.float32),
            pltpu.SemaphoreType.DMA,
            pltpu.SemaphoreType.DMA,
        ],
    )
    def body(ids_hbm, table_hbm, out_hbm, idx_v, buf0, buf1, sem0, sem1):
        wid = lax.axis_index("s") * NUM_CORES + lax.axis_index("c")
        pltpu.sync_copy(ids_hbm.at[wid], idx_v)

        bufs = (buf0, buf1)
        sems = (sem0, sem1)

        pltpu.async_copy(table_hbm.at[idx_v.at[0]], buf0, sem0)
        pltpu.async_copy(table_hbm.at[idx_v.at[1]], buf1, sem1)

        @pl.loop(0, n_chunks - 2, step=2)
        def _(g):
            for b in range(2):
                j = g + b
                pltpu.make_async_copy(
                    table_hbm.at[idx_v.at[j]], bufs[b], sems[b]
                ).wait()
                pltpu.sync_copy(
                    bufs[b].at[:, pl.ds(0, EMBED_DIM)], out_hbm.at[wid, j]
                )
                pltpu.async_copy(table_hbm.at[idx_v.at[j + 2]], bufs[b], sems[b])

        for b in range(2):
            j = n_chunks - 2 + b
            pltpu.make_async_copy(
                table_hbm.at[idx_v.at[j]], bufs[b], sems[b]
            ).wait()
            pltpu.sync_copy(
                bufs[b].at[:, pl.ds(0, EMBED_DIM)], out_hbm.at[wid, j]
            )

    return body(ids, table_p)


def kernel(token_ids, table):
    batch, seq = token_ids.shape
    total = batch * seq
    assert total % (NUM_WORKERS * CHUNK) == 0
    n_chunks = total // (NUM_WORKERS * CHUNK)
    ids = token_ids.reshape(NUM_WORKERS, n_chunks, CHUNK).astype(jnp.int32)
    table_p = _tc_stage(table.T)
    out = _lookup(ids, table_p, n_chunks)
    return out.reshape(batch, seq, EMBED_DIM)


# speedup vs baseline: 2.2154x; 1.1736x over previous
import functools
import math

import jax
import jax.numpy as jnp
from jax import lax
from jax.experimental import pallas as pl
from jax.experimental.pallas import tpu as pltpu
from jax.experimental.pallas import tpu_sc as plsc

EMBED_DIM = 64
PADDED_DIM = 128
NUM_CORES = 2
NUM_SUBCORES = 16
NUM_WORKERS = NUM_CORES * NUM_SUBCORES
CHUNK = 128
TBLK = 16384

_mesh = plsc.VectorSubcoreMesh(
    core_axis_name="c",
    subcore_axis_name="s",
    num_cores=NUM_CORES,
    num_subcores=NUM_SUBCORES,
)


def _tc_stage(tab_t):
    d, v = tab_t.shape

    def body(in_ref, out_ref):
        out_ref[:, 0:EMBED_DIM] = in_ref[...].T

    return pl.pallas_call(
        body,
        grid=(math.ceil(v / TBLK),),
        in_specs=[pl.BlockSpec((d, TBLK), lambda g: (0, g))],
        out_specs=pl.BlockSpec((TBLK, PADDED_DIM), lambda g: (g, 0)),
        out_shape=jax.ShapeDtypeStruct((v, PADDED_DIM), jnp.float32),
    )(tab_t)


SEQ_PAD = 56


@functools.partial(jax.jit, static_argnames=("seq", "batch"))
def _lookup(ids_t, table_p, seq, batch):
    @functools.partial(
        pl.kernel,
        out_type=jax.ShapeDtypeStruct((batch, SEQ_PAD, PADDED_DIM), jnp.float32),
        mesh=_mesh,
        compiler_params=pltpu.CompilerParams(use_tc_tiling_on_sc=False),
        scratch_types=[
            pltpu.VMEM((seq, CHUNK), jnp.int32),
            pltpu.VMEM((CHUNK, PADDED_DIM), jnp.float32),
            pltpu.VMEM((CHUNK, PADDED_DIM), jnp.float32),
            pltpu.SemaphoreType.DMA,
            pltpu.SemaphoreType.DMA,
        ],
    )
    def body(ids_hbm, table_hbm, out_hbm, idx_v, buf0, buf1, sem0, sem1):
        wid = lax.axis_index("s") * NUM_CORES + lax.axis_index("c")
        b0 = wid * CHUNK
        pltpu.sync_copy(ids_hbm.at[:, pl.ds(b0, CHUNK)], idx_v)

        bufs = (buf0, buf1)
        sems = (sem0, sem1)

        pltpu.async_copy(table_hbm.at[idx_v.at[0]], buf0, sem0)
        pltpu.async_copy(table_hbm.at[idx_v.at[1]], buf1, sem1)

        @pl.loop(0, seq - 2, step=2)
        def _(g):
            for b in range(2):
                s = g + b
                pltpu.make_async_copy(
                    table_hbm.at[idx_v.at[s]], bufs[b], sems[b]
                ).wait()
                pltpu.sync_copy(bufs[b], out_hbm.at[pl.ds(b0, CHUNK), s])
                pltpu.async_copy(table_hbm.at[idx_v.at[s + 2]], bufs[b], sems[b])

        for b in range(2):
            s = seq - 2 + b
            pltpu.make_async_copy(
                table_hbm.at[idx_v.at[s]], bufs[b], sems[b]
            ).wait()
            pltpu.sync_copy(bufs[b], out_hbm.at[pl.ds(b0, CHUNK), s])

    return body(ids_t, table_p)


def kernel(token_ids, table):
    batch, seq = token_ids.shape
    assert batch == NUM_WORKERS * CHUNK and seq % 2 == 0 and seq <= SEQ_PAD
    ids_t = token_ids.T.astype(jnp.int32)
    table_p = _tc_stage(table.T)
    out6 = _lookup(ids_t, table_p, seq, batch)
    return out6[:, :seq, :EMBED_DIM]


# speedup vs baseline: 2.2978x; 1.0372x over previous
import functools
import math

import jax
import jax.numpy as jnp
from jax import lax
from jax.experimental import pallas as pl
from jax.experimental.pallas import tpu as pltpu
from jax.experimental.pallas import tpu_sc as plsc

EMBED_DIM = 64
PADDED_DIM = 128
NUM_CORES = 2
NUM_SUBCORES = 16
NUM_WORKERS = NUM_CORES * NUM_SUBCORES
CHUNK = 128
TBLK = 16384

_mesh = plsc.VectorSubcoreMesh(
    core_axis_name="c",
    subcore_axis_name="s",
    num_cores=NUM_CORES,
    num_subcores=NUM_SUBCORES,
)


def _tc_stage(tab_t):
    d, v = tab_t.shape

    def body(in_ref, out_ref):
        out_ref[:, 0:EMBED_DIM] = in_ref[...].T

    return pl.pallas_call(
        body,
        grid=(math.ceil(v / TBLK),),
        in_specs=[pl.BlockSpec((d, TBLK), lambda g: (0, g))],
        out_specs=pl.BlockSpec((TBLK, PADDED_DIM), lambda g: (g, 0)),
        out_shape=jax.ShapeDtypeStruct((v, PADDED_DIM), jnp.float32),
    )(tab_t)


SEQ_PAD = 56


@functools.partial(jax.jit, static_argnames=("seq", "batch"))
def _lookup(ids_t, table_p, seq, batch):
    @functools.partial(
        pl.kernel,
        out_type=jax.ShapeDtypeStruct((batch, SEQ_PAD, PADDED_DIM), jnp.float32),
        mesh=_mesh,
        compiler_params=pltpu.CompilerParams(use_tc_tiling_on_sc=False),
        scratch_types=[
            pltpu.VMEM((seq, CHUNK), jnp.int32),
            pltpu.VMEM((CHUNK, PADDED_DIM), jnp.float32),
            pltpu.VMEM((CHUNK, PADDED_DIM), jnp.float32),
            pltpu.SemaphoreType.DMA,
            pltpu.SemaphoreType.DMA,
        ],
    )
    def body(ids_hbm, table_hbm, out_hbm, idx_v, buf0, buf1, sem0, sem1):
        wid = lax.axis_index("s") * NUM_CORES + lax.axis_index("c")
        b0 = wid * CHUNK
        pltpu.sync_copy(ids_hbm.at[:, pl.ds(b0, CHUNK)], idx_v)

        bufs = (buf0, buf1)
        sems = (sem0, sem1)

        pltpu.async_copy(table_hbm.at[idx_v.at[0]], buf0, sem0)
        pltpu.async_copy(table_hbm.at[idx_v.at[1]], buf1, sem1)

        @pl.loop(0, seq - 2, step=2)
        def _(g):
            for b in range(2):
                s = g + b
                pltpu.make_async_copy(
                    table_hbm.at[idx_v.at[s]], bufs[b], sems[b]
                ).wait()
                pltpu.sync_copy(
                    bufs[b].at[:, pl.ds(0, EMBED_DIM)],
                    out_hbm.at[pl.ds(b0, CHUNK), s, pl.ds(0, EMBED_DIM)],
                )
                pltpu.async_copy(table_hbm.at[idx_v.at[s + 2]], bufs[b], sems[b])

        for b in range(2):
            s = seq - 2 + b
            pltpu.make_async_copy(
                table_hbm.at[idx_v.at[s]], bufs[b], sems[b]
            ).wait()
            pltpu.sync_copy(
                    bufs[b].at[:, pl.ds(0, EMBED_DIM)],
                    out_hbm.at[pl.ds(b0, CHUNK), s, pl.ds(0, EMBED_DIM)],
                )

    return body(ids_t, table_p)


def kernel(token_ids, table):
    batch, seq = token_ids.shape
    assert batch == NUM_WORKERS * CHUNK and seq % 2 == 0 and seq <= SEQ_PAD
    ids_t = token_ids.T.astype(jnp.int32)
    table_p = _tc_stage(table.T)
    out6 = _lookup(ids_t, table_p, seq, batch)
    return out6[:, :seq, :EMBED_DIM]


# speedup vs baseline: 2.3142x; 1.0071x over previous
import functools
import math

import jax
import jax.numpy as jnp
from jax import lax
from jax.experimental import pallas as pl
from jax.experimental.pallas import tpu as pltpu
from jax.experimental.pallas import tpu_sc as plsc

EMBED_DIM = 64
PADDED_DIM = 128
NUM_CORES = 2
NUM_SUBCORES = 16
NUM_WORKERS = NUM_CORES * NUM_SUBCORES
CHUNK = 128
TBLK = 20480

_mesh = plsc.VectorSubcoreMesh(
    core_axis_name="c",
    subcore_axis_name="s",
    num_cores=NUM_CORES,
    num_subcores=NUM_SUBCORES,
)


def _tc_stage(tab_t):
    d, v = tab_t.shape

    def body(in_ref, out_ref):
        out_ref[:, 0:EMBED_DIM] = in_ref[...].T

    return pl.pallas_call(
        body,
        grid=(math.ceil(v / TBLK),),
        in_specs=[pl.BlockSpec((d, TBLK), lambda g: (0, g))],
        out_specs=pl.BlockSpec((TBLK, PADDED_DIM), lambda g: (g, 0)),
        out_shape=jax.ShapeDtypeStruct((v, PADDED_DIM), jnp.float32),
    )(tab_t)


SEQ_PAD = 56


@functools.partial(jax.jit, static_argnames=("seq", "batch"))
def _lookup(ids_t, table_p, seq, batch):
    @functools.partial(
        pl.kernel,
        out_type=jax.ShapeDtypeStruct((batch, SEQ_PAD, PADDED_DIM), jnp.float32),
        mesh=_mesh,
        compiler_params=pltpu.CompilerParams(use_tc_tiling_on_sc=False),
        scratch_types=[
            pltpu.VMEM((seq, CHUNK), jnp.int32),
            pltpu.VMEM((CHUNK, PADDED_DIM), jnp.float32),
            pltpu.VMEM((CHUNK, PADDED_DIM), jnp.float32),
            pltpu.SemaphoreType.DMA,
            pltpu.SemaphoreType.DMA,
        ],
    )
    def body(ids_hbm, table_hbm, out_hbm, idx_v, buf0, buf1, sem0, sem1):
        wid = lax.axis_index("s") * NUM_CORES + lax.axis_index("c")
        b0 = wid * CHUNK
        pltpu.sync_copy(ids_hbm.at[:, pl.ds(b0, CHUNK)], idx_v)

        bufs = (buf0, buf1)
        sems = (sem0, sem1)

        pltpu.async_copy(table_hbm.at[idx_v.at[0]], buf0, sem0)
        pltpu.async_copy(table_hbm.at[idx_v.at[1]], buf1, sem1)

        @pl.loop(0, seq - 2, step=2)
        def _(g):
            for b in range(2):
                s = g + b
                pltpu.make_async_copy(
                    table_hbm.at[idx_v.at[s]], bufs[b], sems[b]
                ).wait()
                pltpu.sync_copy(
                    bufs[b].at[:, pl.ds(0, EMBED_DIM)],
                    out_hbm.at[pl.ds(b0, CHUNK), s, pl.ds(0, EMBED_DIM)],
                )
                pltpu.async_copy(table_hbm.at[idx_v.at[s + 2]], bufs[b], sems[b])

        for b in range(2):
            s = seq - 2 + b
            pltpu.make_async_copy(
                table_hbm.at[idx_v.at[s]], bufs[b], sems[b]
            ).wait()
            pltpu.sync_copy(
                    bufs[b].at[:, pl.ds(0, EMBED_DIM)],
                    out_hbm.at[pl.ds(b0, CHUNK), s, pl.ds(0, EMBED_DIM)],
                )

    return body(ids_t, table_p)


def kernel(token_ids, table):
    batch, seq = token_ids.shape
    assert batch == NUM_WORKERS * CHUNK and seq % 2 == 0 and seq <= SEQ_PAD
    ids_t = token_ids.T.astype(jnp.int32)
    table_p = _tc_stage(table.T)
    out6 = _lookup(ids_t, table_p, seq, batch)
    return out6[:, :seq, :EMBED_DIM]
